# baseline (speedup 1.0000x reference)
import functools

import jax
import jax.numpy as jnp
from jax import lax
from jax.experimental import pallas as pl
from jax.experimental.pallas import tpu as pltpu
from jax.experimental.pallas import tpu_sc as plsc

_NC = 2
_NS = 16
_NW = _NC * _NS

_CH = 64
_NCHUNK = 4


def _sc_gather(table, flat_ids, off, n_rows):
    d = table.shape[1]
    b_per_w = n_rows // _NW
    n_ch = b_per_w // _CH
    mesh = plsc.VectorSubcoreMesh(core_axis_name="c", subcore_axis_name="s")

    @functools.partial(
        pl.kernel,
        out_type=jax.ShapeDtypeStruct((n_rows, d), jnp.float32),
        mesh=mesh,
        scratch_types=[
            pltpu.VMEM((b_per_w,), jnp.int32),
            pltpu.VMEM((_CH, d), jnp.float32),
            pltpu.VMEM((_CH, d), jnp.float32),
            pltpu.SemaphoreType.DMA,
            pltpu.SemaphoreType.DMA,
        ],
    )
    def gather_kernel(table_hbm, idx_hbm, out_hbm, idx_v, buf0, buf1, sem0, sem1):
        wid = lax.axis_index("s") * _NC + lax.axis_index("c")
        base = wid * b_per_w
        pltpu.sync_copy(idx_hbm.at[pl.ds(off + base, b_per_w)], idx_v)

        bufs = (buf0, buf1)
        sems = (sem0, sem1)
        copies = [None] * n_ch
        copies[0] = pltpu.async_copy(
            table_hbm.at[idx_v.at[pl.ds(0, _CH)]], bufs[0], sems[0]
        )
        for ci in range(n_ch):
            if ci + 1 < n_ch:
                copies[ci + 1] = pltpu.async_copy(
                    table_hbm.at[idx_v.at[pl.ds((ci + 1) * _CH, _CH)]],
                    bufs[(ci + 1) % 2],
                    sems[(ci + 1) % 2],
                )
            copies[ci].wait()
            pltpu.sync_copy(bufs[ci % 2], out_hbm.at[pl.ds(base + ci * _CH, _CH)])

    return gather_kernel(table, flat_ids)


def _fused_body(g_ref, tt_ref, pos_ref, ttab_ref, gam_ref, bet_ref, *rest):
    o_ref = rest[-1]
    x = g_ref[...] + pos_ref[...]
    t = tt_ref[...]
    x = x + jnp.where(t == 0, ttab_ref[0:1, :], ttab_ref[1:2, :])
    mean = jnp.mean(x, axis=1, keepdims=True)
    c = x - mean
    var = jnp.mean(c * c, axis=1, keepdims=True)
    y = c * lax.rsqrt(var + 1e-12)
    o_ref[...] = y * gam_ref[...] + bet_ref[...]


def _tc_fuse_chunk(gathered_c, tt_all, pos, ttab, gamma, beta, out_prev,
                   chunk_idx, n_total, chunk_batches, s_blocks, blk):
    d = gathered_c.shape[1]
    cb = chunk_idx * chunk_batches
    in_specs = [
        pl.BlockSpec((blk, d), lambda j, b: (b * s_blocks + j, 0)),
        pl.BlockSpec((blk, 1), lambda j, b: ((cb + b) * s_blocks + j, 0)),
        pl.BlockSpec((blk, d), lambda j, b: (j, 0)),
        pl.BlockSpec(ttab.shape, lambda j, b: (0, 0)),
        pl.BlockSpec((1, d), lambda j, b: (0, 0)),
        pl.BlockSpec((1, d), lambda j, b: (0, 0)),
    ]
    args = [gathered_c, tt_all, pos, ttab, gamma, beta]
    aliases = {}
    if out_prev is not None:
        in_specs.append(pl.BlockSpec((8, 128), lambda j, b: (0, 0)))
        args.append(out_prev)
        aliases = {6: 0}
    return pl.pallas_call(
        _fused_body,
        grid=(s_blocks, chunk_batches),
        in_specs=in_specs,
        out_specs=pl.BlockSpec(
            (blk, d), lambda j, b: ((cb + b) * s_blocks + j, 0)),
        out_shape=jax.ShapeDtypeStruct((n_total, d), jnp.float32),
        input_output_aliases=aliases,
    )(*args)


def kernel(input_ids, token_type_ids, token_embedding, position_embedding,
           token_type_embedding, ln_gamma, ln_beta):
    b, s = input_ids.shape
    d = token_embedding.shape[1]
    n = b * s
    blk = 1024
    s_blocks = s // blk
    chunk_batches = b // _NCHUNK
    rows_per_chunk = n // _NCHUNK

    flat_ids = input_ids.reshape(n).astype(jnp.int32)
    tt = token_type_ids.reshape(n, 1).astype(jnp.int32)
    gamma = ln_gamma.reshape(1, d)
    beta = ln_beta.reshape(1, d)

    gathered = [
        _sc_gather(token_embedding, flat_ids, c * rows_per_chunk, rows_per_chunk)
        for c in range(_NCHUNK)
    ]
    out = None
    for c in range(_NCHUNK):
        out = _tc_fuse_chunk(
            gathered[c], tt, position_embedding, token_type_embedding,
            gamma, beta, out, c, n, chunk_batches, s_blocks, blk)
    return out.reshape(b, s, d)


# speedup vs baseline: 1.0645x; 1.0645x over previous
import functools

import jax
import jax.numpy as jnp
from jax import lax
from jax.experimental import pallas as pl
from jax.experimental.pallas import tpu as pltpu
from jax.experimental.pallas import tpu_sc as plsc

_NC = 2
_NS = 16
_NW = _NC * _NS

_CH = 64
_CHUNK_BATCHES = (3, 1)


def _sc_gather(table, flat_ids, off, n_rows):
    d = table.shape[1]
    b_per_w = n_rows // _NW
    n_ch = b_per_w // _CH
    mesh = plsc.VectorSubcoreMesh(core_axis_name="c", subcore_axis_name="s")

    @functools.partial(
        pl.kernel,
        out_type=jax.ShapeDtypeStruct((n_rows, d), jnp.float32),
        mesh=mesh,
        scratch_types=[
            pltpu.VMEM((b_per_w,), jnp.int32),
            pltpu.VMEM((_CH, d), jnp.float32),
            pltpu.VMEM((_CH, d), jnp.float32),
            pltpu.SemaphoreType.DMA,
            pltpu.SemaphoreType.DMA,
        ],
    )
    def gather_kernel(table_hbm, idx_hbm, out_hbm, idx_v, buf0, buf1, sem0, sem1):
        wid = lax.axis_index("s") * _NC + lax.axis_index("c")
        base = wid * b_per_w
        pltpu.sync_copy(idx_hbm.at[pl.ds(off + base, b_per_w)], idx_v)

        bufs = (buf0, buf1)
        sems = (sem0, sem1)
        copies = [None] * n_ch
        copies[0] = pltpu.async_copy(
            table_hbm.at[idx_v.at[pl.ds(0, _CH)]], bufs[0], sems[0]
        )
        for ci in range(n_ch):
            if ci + 1 < n_ch:
                copies[ci + 1] = pltpu.async_copy(
                    table_hbm.at[idx_v.at[pl.ds((ci + 1) * _CH, _CH)]],
                    bufs[(ci + 1) % 2],
                    sems[(ci + 1) % 2],
                )
            copies[ci].wait()
            pltpu.sync_copy(bufs[ci % 2], out_hbm.at[pl.ds(base + ci * _CH, _CH)])

    return gather_kernel(table, flat_ids)


def _fused_body(g_ref, tt_ref, pos_ref, ttab_ref, gam_ref, bet_ref, *rest):
    o_ref = rest[-1]
    x = g_ref[...] + pos_ref[...]
    t = tt_ref[...]
    x = x + jnp.where(t == 0, ttab_ref[0:1, :], ttab_ref[1:2, :])
    mean = jnp.mean(x, axis=1, keepdims=True)
    c = x - mean
    var = jnp.mean(c * c, axis=1, keepdims=True)
    y = c * lax.rsqrt(var + 1e-12)
    o_ref[...] = y * gam_ref[...] + bet_ref[...]


def _tc_fuse_chunk(gathered_c, tt_all, pos, ttab, gamma, beta, out_prev,
                   cb, n_total, chunk_batches, s_blocks, blk):
    d = gathered_c.shape[1]
    in_specs = [
        pl.BlockSpec((blk, d), lambda j, b: (b * s_blocks + j, 0)),
        pl.BlockSpec((blk, 1), lambda j, b: ((cb + b) * s_blocks + j, 0)),
        pl.BlockSpec((blk, d), lambda j, b: (j, 0)),
        pl.BlockSpec(ttab.shape, lambda j, b: (0, 0)),
        pl.BlockSpec((1, d), lambda j, b: (0, 0)),
        pl.BlockSpec((1, d), lambda j, b: (0, 0)),
    ]
    args = [gathered_c, tt_all, pos, ttab, gamma, beta]
    aliases = {}
    if out_prev is not None:
        in_specs.append(pl.BlockSpec((8, 128), lambda j, b: (0, 0)))
        args.append(out_prev)
        aliases = {6: 0}
    return pl.pallas_call(
        _fused_body,
        grid=(s_blocks, chunk_batches),
        in_specs=in_specs,
        out_specs=pl.BlockSpec(
            (blk, d), lambda j, b: ((cb + b) * s_blocks + j, 0)),
        out_shape=jax.ShapeDtypeStruct((n_total, d), jnp.float32),
        input_output_aliases=aliases,
    )(*args)


def kernel(input_ids, token_type_ids, token_embedding, position_embedding,
           token_type_embedding, ln_gamma, ln_beta):
    b, s = input_ids.shape
    d = token_embedding.shape[1]
    n = b * s
    blk = 1024
    s_blocks = s // blk

    flat_ids = input_ids.reshape(n).astype(jnp.int32)
    tt = token_type_ids.reshape(n, 1).astype(jnp.int32)
    gamma = ln_gamma.reshape(1, d)
    beta = ln_beta.reshape(1, d)

    batch_offs = [sum(_CHUNK_BATCHES[:i]) for i in range(len(_CHUNK_BATCHES))]
    gathered = [
        _sc_gather(token_embedding, flat_ids, cb * s, nb * s)
        for cb, nb in zip(batch_offs, _CHUNK_BATCHES)
    ]
    out = None
    for g, cb, nb in zip(gathered, batch_offs, _CHUNK_BATCHES):
        out = _tc_fuse_chunk(
            g, tt, position_embedding, token_type_embedding,
            gamma, beta, out, cb, n, nb, s_blocks, blk)
    return out.reshape(b, s, d)


# speedup vs baseline: 1.0705x; 1.0056x over previous
import functools

import jax
import jax.numpy as jnp
from jax import lax
from jax.experimental import pallas as pl
from jax.experimental.pallas import tpu as pltpu
from jax.experimental.pallas import tpu_sc as plsc

_NC = 2
_NS = 16
_NW = _NC * _NS

_CH = 32
_CHUNK_BATCHES = (2, 2)


def _sc_gather(table, flat_ids, off, n_rows):
    d = table.shape[1]
    b_per_w = n_rows // _NW
    n_ch = b_per_w // _CH
    mesh = plsc.VectorSubcoreMesh(core_axis_name="c", subcore_axis_name="s")

    @functools.partial(
        pl.kernel,
        out_type=jax.ShapeDtypeStruct((n_rows, d), jnp.float32),
        mesh=mesh,
        scratch_types=[
            pltpu.VMEM((b_per_w,), jnp.int32),
            pltpu.VMEM((_CH, d), jnp.float32),
            pltpu.VMEM((_CH, d), jnp.float32),
            pltpu.SemaphoreType.DMA,
            pltpu.SemaphoreType.DMA,
        ],
    )
    def gather_kernel(table_hbm, idx_hbm, out_hbm, idx_v, buf0, buf1, sem0, sem1):
        wid = lax.axis_index("s") * _NC + lax.axis_index("c")
        base = wid * b_per_w
        pltpu.sync_copy(idx_hbm.at[pl.ds(off + base, b_per_w)], idx_v)

        bufs = (buf0, buf1)
        sems = (sem0, sem1)
        copies = [None] * n_ch
        copies[0] = pltpu.async_copy(
            table_hbm.at[idx_v.at[pl.ds(0, _CH)]], bufs[0], sems[0]
        )
        for ci in range(n_ch):
            if ci + 1 < n_ch:
                copies[ci + 1] = pltpu.async_copy(
                    table_hbm.at[idx_v.at[pl.ds((ci + 1) * _CH, _CH)]],
                    bufs[(ci + 1) % 2],
                    sems[(ci + 1) % 2],
                )
            copies[ci].wait()
            pltpu.sync_copy(bufs[ci % 2], out_hbm.at[pl.ds(base + ci * _CH, _CH)])

    return gather_kernel(table, flat_ids)


def _fused_body(g_ref, tt_ref, pos_ref, ttab_ref, gam_ref, bet_ref, *rest):
    o_ref = rest[-1]
    x = g_ref[...] + pos_ref[...]
    t = tt_ref[...]
    x = x + jnp.where(t == 0, ttab_ref[0:1, :], ttab_ref[1:2, :])
    mean = jnp.mean(x, axis=1, keepdims=True)
    c = x - mean
    var = jnp.mean(c * c, axis=1, keepdims=True)
    y = c * lax.rsqrt(var + 1e-12)
    o_ref[...] = y * gam_ref[...] + bet_ref[...]


def _tc_fuse_chunk(gathered_c, tt_all, pos, ttab, gamma, beta, out_prev,
                   cb, n_total, chunk_batches, s_blocks, blk):
    d = gathered_c.shape[1]
    in_specs = [
        pl.BlockSpec((blk, d), lambda j, b: (b * s_blocks + j, 0)),
        pl.BlockSpec((blk, 1), lambda j, b: ((cb + b) * s_blocks + j, 0)),
        pl.BlockSpec((blk, d), lambda j, b: (j, 0)),
        pl.BlockSpec(ttab.shape, lambda j, b: (0, 0)),
        pl.BlockSpec((1, d), lambda j, b: (0, 0)),
        pl.BlockSpec((1, d), lambda j, b: (0, 0)),
    ]
    args = [gathered_c, tt_all, pos, ttab, gamma, beta]
    aliases = {}
    if out_prev is not None:
        in_specs.append(pl.BlockSpec((8, 128), lambda j, b: (0, 0)))
        args.append(out_prev)
        aliases = {6: 0}
    return pl.pallas_call(
        _fused_body,
        grid=(s_blocks, chunk_batches),
        in_specs=in_specs,
        out_specs=pl.BlockSpec(
            (blk, d), lambda j, b: ((cb + b) * s_blocks + j, 0)),
        out_shape=jax.ShapeDtypeStruct((n_total, d), jnp.float32),
        input_output_aliases=aliases,
    )(*args)


def kernel(input_ids, token_type_ids, token_embedding, position_embedding,
           token_type_embedding, ln_gamma, ln_beta):
    b, s = input_ids.shape
    d = token_embedding.shape[1]
    n = b * s
    blk = 1024
    s_blocks = s // blk

    flat_ids = input_ids.reshape(n).astype(jnp.int32)
    tt = token_type_ids.reshape(n, 1).astype(jnp.int32)
    gamma = ln_gamma.reshape(1, d)
    beta = ln_beta.reshape(1, d)

    batch_offs = [sum(_CHUNK_BATCHES[:i]) for i in range(len(_CHUNK_BATCHES))]
    gathered = [
        _sc_gather(token_embedding, flat_ids, cb * s, nb * s)
        for cb, nb in zip(batch_offs, _CHUNK_BATCHES)
    ]
    out = None
    for g, cb, nb in zip(gathered, batch_offs, _CHUNK_BATCHES):
        out = _tc_fuse_chunk(
            g, tt, position_embedding, token_type_embedding,
            gamma, beta, out, cb, n, nb, s_blocks, blk)
    return out.reshape(b, s, d)


# speedup vs baseline: 1.0798x; 1.0087x over previous
import functools

import jax
import jax.numpy as jnp
from jax import lax
from jax.experimental import pallas as pl
from jax.experimental.pallas import tpu as pltpu
from jax.experimental.pallas import tpu_sc as plsc

_NC = 2
_NS = 16
_NW = _NC * _NS

_CH = 64
_CHUNK_BATCHES = (2, 2)


def _sc_gather(table, flat_ids, off, n_rows):
    d = table.shape[1]
    b_per_w = n_rows // _NW
    n_ch = b_per_w // _CH
    mesh = plsc.VectorSubcoreMesh(core_axis_name="c", subcore_axis_name="s")

    @functools.partial(
        pl.kernel,
        out_type=jax.ShapeDtypeStruct((n_rows, d), jnp.float32),
        mesh=mesh,
        scratch_types=[
            pltpu.VMEM((b_per_w,), jnp.int32),
            pltpu.VMEM((_CH, d), jnp.float32),
            pltpu.VMEM((_CH, d), jnp.float32),
            pltpu.SemaphoreType.DMA,
            pltpu.SemaphoreType.DMA,
        ],
    )
    def gather_kernel(table_hbm, idx_hbm, out_hbm, idx_v, buf0, buf1, sem0, sem1):
        wid = lax.axis_index("s") * _NC + lax.axis_index("c")
        base = wid * b_per_w
        pltpu.sync_copy(idx_hbm.at[pl.ds(off + base, b_per_w)], idx_v)

        bufs = (buf0, buf1)
        sems = (sem0, sem1)
        copies = [None] * n_ch
        copies[0] = pltpu.async_copy(
            table_hbm.at[idx_v.at[pl.ds(0, _CH)]], bufs[0], sems[0]
        )
        for ci in range(n_ch):
            if ci + 1 < n_ch:
                copies[ci + 1] = pltpu.async_copy(
                    table_hbm.at[idx_v.at[pl.ds((ci + 1) * _CH, _CH)]],
                    bufs[(ci + 1) % 2],
                    sems[(ci + 1) % 2],
                )
            copies[ci].wait()
            pltpu.sync_copy(bufs[ci % 2], out_hbm.at[pl.ds(base + ci * _CH, _CH)])

    return gather_kernel(table, flat_ids)


def _fused_body(g_ref, tt_ref, pos_ref, ttab_ref, gam_ref, bet_ref, *rest):
    o_ref = rest[-1]
    x = g_ref[...] + pos_ref[...]
    t = tt_ref[...]
    x = x + jnp.where(t == 0, ttab_ref[0:1, :], ttab_ref[1:2, :])
    mean = jnp.mean(x, axis=1, keepdims=True)
    c = x - mean
    var = jnp.mean(c * c, axis=1, keepdims=True)
    y = c * lax.rsqrt(var + 1e-12)
    o_ref[...] = y * gam_ref[...] + bet_ref[...]


def _tc_fuse_chunk(gathered_c, tt_all, pos, ttab, gamma, beta, out_prev,
                   cb, n_total, chunk_batches, s_blocks, blk):
    d = gathered_c.shape[1]
    in_specs = [
        pl.BlockSpec((blk, d), lambda j, b: (b * s_blocks + j, 0)),
        pl.BlockSpec((blk, 1), lambda j, b: ((cb + b) * s_blocks + j, 0)),
        pl.BlockSpec((blk, d), lambda j, b: (j, 0)),
        pl.BlockSpec(ttab.shape, lambda j, b: (0, 0)),
        pl.BlockSpec((1, d), lambda j, b: (0, 0)),
        pl.BlockSpec((1, d), lambda j, b: (0, 0)),
    ]
    args = [gathered_c, tt_all, pos, ttab, gamma, beta]
    aliases = {}
    if out_prev is not None:
        in_specs.append(pl.BlockSpec((8, 128), lambda j, b: (0, 0)))
        args.append(out_prev)
        aliases = {6: 0}
    return pl.pallas_call(
        _fused_body,
        grid=(s_blocks, chunk_batches),
        in_specs=in_specs,
        out_specs=pl.BlockSpec(
            (blk, d), lambda j, b: ((cb + b) * s_blocks + j, 0)),
        out_shape=jax.ShapeDtypeStruct((n_total, d), jnp.float32),
        input_output_aliases=aliases,
    )(*args)


def kernel(input_ids, token_type_ids, token_embedding, position_embedding,
           token_type_embedding, ln_gamma, ln_beta):
    b, s = input_ids.shape
    d = token_embedding.shape[1]
    n = b * s
    blk = 1024
    s_blocks = s // blk

    flat_ids = input_ids.reshape(n).astype(jnp.int32)
    tt = token_type_ids.reshape(n, 1).astype(jnp.int32)
    gamma = ln_gamma.reshape(1, d)
    beta = ln_beta.reshape(1, d)

    batch_offs = [sum(_CHUNK_BATCHES[:i]) for i in range(len(_CHUNK_BATCHES))]
    gathered = [
        _sc_gather(token_embedding, flat_ids, cb * s, nb * s)
        for cb, nb in zip(batch_offs, _CHUNK_BATCHES)
    ]
    out = None
    for g, cb, nb in zip(gathered, batch_offs, _CHUNK_BATCHES):
        out = _tc_fuse_chunk(
            g, tt, position_embedding, token_type_embedding,
            gamma, beta, out, cb, n, nb, s_blocks, blk)
    return out.reshape(b, s, d)


# speedup vs baseline: 1.1196x; 1.0369x over previous
import functools

import jax
import jax.numpy as jnp
from jax import lax
from jax.experimental import pallas as pl
from jax.experimental.pallas import tpu as pltpu
from jax.experimental.pallas import tpu_sc as plsc

_NC = 2
_NS = 16
_NW = _NC * _NS

_CH = 64
_CHUNK_BATCHES = (2, 2)


def _sc_gather(table, flat_ids, off, n_rows):
    d = table.shape[1]
    b_per_w = n_rows // _NW
    n_ch = b_per_w // _CH
    mesh = plsc.VectorSubcoreMesh(core_axis_name="c", subcore_axis_name="s")

    @functools.partial(
        pl.kernel,
        out_type=jax.ShapeDtypeStruct((n_rows, d), jnp.float32),
        mesh=mesh,
        scratch_types=[
            pltpu.VMEM((b_per_w,), jnp.int32),
            pltpu.VMEM((_CH, d), jnp.float32),
            pltpu.VMEM((_CH, d), jnp.float32),
            pltpu.SemaphoreType.DMA,
            pltpu.SemaphoreType.DMA,
        ],
    )
    def gather_kernel(table_hbm, idx_hbm, out_hbm, idx_v, buf0, buf1, sem0, sem1):
        wid = lax.axis_index("s") * _NC + lax.axis_index("c")
        base = wid * b_per_w
        pltpu.sync_copy(idx_hbm.at[pl.ds(off + base, b_per_w)], idx_v)

        bufs = (buf0, buf1)
        sems = (sem0, sem1)
        copies = [None] * n_ch
        copies[0] = pltpu.async_copy(
            table_hbm.at[idx_v.at[pl.ds(0, _CH)]], bufs[0], sems[0]
        )
        for ci in range(n_ch):
            if ci + 1 < n_ch:
                copies[ci + 1] = pltpu.async_copy(
                    table_hbm.at[idx_v.at[pl.ds((ci + 1) * _CH, _CH)]],
                    bufs[(ci + 1) % 2],
                    sems[(ci + 1) % 2],
                )
            copies[ci].wait()
            pltpu.sync_copy(bufs[ci % 2], out_hbm.at[pl.ds(base + ci * _CH, _CH)])

    return gather_kernel(table, flat_ids)


def _fused_body(g_ref, tt_ref, pos_ref, ttab_ref, gam_ref, bet_ref, *rest):
    o_ref = rest[-1]
    x = g_ref[...] + pos_ref[...]
    t = tt_ref[...]
    x = x + jnp.where(t == 0, ttab_ref[0:1, :], ttab_ref[1:2, :])
    mean = jnp.mean(x, axis=1, keepdims=True)
    c = x - mean
    var = jnp.mean(c * c, axis=1, keepdims=True)
    y = c * lax.rsqrt(var + 1e-12)
    o_ref[...] = y * gam_ref[...] + bet_ref[...]


def _tc_fuse_chunk(gathered_c, tt_all, pos, ttab, gamma, beta, out_prev,
                   cb, n_total, chunk_batches, s_blocks, blk):
    d = gathered_c.shape[1]
    in_specs = [
        pl.BlockSpec((blk, d), lambda j, b: (b * s_blocks + j, 0)),
        pl.BlockSpec((blk, 1), lambda j, b: ((cb + b) * s_blocks + j, 0)),
        pl.BlockSpec((blk, d), lambda j, b: (j, 0)),
        pl.BlockSpec(ttab.shape, lambda j, b: (0, 0)),
        pl.BlockSpec((1, d), lambda j, b: (0, 0)),
        pl.BlockSpec((1, d), lambda j, b: (0, 0)),
    ]
    args = [gathered_c, tt_all, pos, ttab, gamma, beta]
    aliases = {}
    if out_prev is not None:
        in_specs.append(pl.BlockSpec((8, 128), lambda j, b: (0, 0)))
        args.append(out_prev)
        aliases = {6: 0}
    return pl.pallas_call(
        _fused_body,
        grid=(s_blocks, chunk_batches),
        in_specs=in_specs,
        out_specs=pl.BlockSpec(
            (blk, d), lambda j, b: ((cb + b) * s_blocks + j, 0)),
        out_shape=jax.ShapeDtypeStruct((n_total, d), jnp.float32),
        input_output_aliases=aliases,
    )(*args)


def kernel(input_ids, token_type_ids, token_embedding, position_embedding,
           token_type_embedding, ln_gamma, ln_beta):
    b, s = input_ids.shape
    d = token_embedding.shape[1]
    n = b * s
    blk = 2048
    s_blocks = s // blk

    flat_ids = input_ids.reshape(n).astype(jnp.int32)
    tt = token_type_ids.reshape(n, 1).astype(jnp.int32)
    gamma = ln_gamma.reshape(1, d)
    beta = ln_beta.reshape(1, d)

    batch_offs = [sum(_CHUNK_BATCHES[:i]) for i in range(len(_CHUNK_BATCHES))]
    gathered = [
        _sc_gather(token_embedding, flat_ids, cb * s, nb * s)
        for cb, nb in zip(batch_offs, _CHUNK_BATCHES)
    ]
    out = None
    for g, cb, nb in zip(gathered, batch_offs, _CHUNK_BATCHES):
        out = _tc_fuse_chunk(
            g, tt, position_embedding, token_type_embedding,
            gamma, beta, out, cb, n, nb, s_blocks, blk)
    return out.reshape(b, s, d)
